# initial kernel scaffold (unmeasured)
import jax
import jax.numpy as jnp
from jax import lax
from jax.experimental import pallas as pl
from jax.experimental.pallas import tpu as pltpu

N_DEV = 4
M = 2048
D = 2048
F = 8192
CHUNK = D // N_DEV


def kernel(x, dy):
    x = x.astype(jnp.bfloat16)
    dy = dy.astype(jnp.bfloat16)

    def body(x_ref, dy_ref, out_ref, send_buf, recv_buf, send_sems, recv_sems):
        my = lax.axis_index("i")
        left = lax.rem(my + N_DEV - 1, N_DEV)
        right = lax.rem(my + 1, N_DEV)

        barrier = pltpu.get_barrier_semaphore()
        for nbr in (left, right):
            pl.semaphore_signal(
                barrier, inc=1,
                device_id=(nbr,), device_id_type=pl.DeviceIdType.MESH,
            )
        pl.semaphore_wait(barrier, 2)

        def partial(c):
            xs = x_ref[:, pl.ds(c * CHUNK, CHUNK)]
            return lax.dot_general(
                xs, dy_ref[:, :],
                dimension_numbers=(((0,), (0,)), ((), ())),
                preferred_element_type=jnp.float32,
            )

        for s in range(N_DEV - 1):
            c_send = lax.rem(my + 2 * N_DEV - 1 - s, N_DEV)
            acc = partial(c_send)
            if s > 0:
                acc = acc + recv_buf[s - 1].astype(jnp.float32)
            send_buf[0] = acc.astype(jnp.bfloat16)
            rdma = pltpu.make_async_remote_copy(
                src_ref=send_buf.at[0],
                dst_ref=recv_buf.at[s],
                send_sem=send_sems.at[s],
                recv_sem=recv_sems.at[s],
                device_id=(right,),
                device_id_type=pl.DeviceIdType.MESH,
            )
            rdma.start()
            rdma.wait()

        out_ref[:, :] = partial(my) + recv_buf[N_DEV - 2].astype(jnp.float32)

    return pl.pallas_call(
        body,
        out_shape=jax.ShapeDtypeStruct((CHUNK, F), jnp.float32),
        in_specs=[
            pl.BlockSpec(memory_space=pltpu.VMEM),
            pl.BlockSpec(memory_space=pltpu.VMEM),
        ],
        out_specs=pl.BlockSpec(memory_space=pltpu.VMEM),
        scratch_shapes=[
            pltpu.VMEM((1, CHUNK, F), jnp.bfloat16),
            pltpu.VMEM((N_DEV - 1, CHUNK, F), jnp.bfloat16),
            pltpu.SemaphoreType.DMA((N_DEV - 1,)),
            pltpu.SemaphoreType.DMA((N_DEV - 1,)),
        ],
        compiler_params=pltpu.CompilerParams(collective_id=0),
    )(x, dy)


# baseline (device time: 419232 ns/iter reference)
import jax
import jax.numpy as jnp
from jax import lax
from jax.experimental import pallas as pl
from jax.experimental.pallas import tpu as pltpu

N_DEV = 4
M = 2048
D = 2048
F = 8192
CHUNK = D // N_DEV
FTILE = 1024
NT = F // FTILE


def kernel(x, dy):
    x = x.astype(jnp.bfloat16)
    dy = dy.astype(jnp.bfloat16)

    def body(x_ref, dy_hbm, out_ref, dy_buf, send_buf, recv_buf,
             dy_sems, send_sems, recv_sems):
        my = lax.axis_index("i")
        left = lax.rem(my + N_DEV - 1, N_DEV)
        right = lax.rem(my + 1, N_DEV)

        barrier = pltpu.get_barrier_semaphore()
        for nbr in (left, right):
            pl.semaphore_signal(
                barrier, inc=1,
                device_id=(nbr,), device_id_type=pl.DeviceIdType.MESH,
            )
        pl.semaphore_wait(barrier, 2)

        def dy_copy(t, slot):
            return pltpu.make_async_copy(
                dy_hbm.at[:, pl.ds(t * FTILE, FTILE)],
                dy_buf.at[slot],
                dy_sems.at[slot],
            )

        dy_copy(0, 0).start()

        for s in range(N_DEV):
            if s < N_DEV - 1:
                c = lax.rem(my + 2 * N_DEV - 1 - s, N_DEV)
            else:
                c = my
            xs = x_ref[:, pl.ds(c * CHUNK, CHUNK)]

            for t in range(NT):
                g = s * NT + t
                slot = g % 2
                if g + 1 < N_DEV * NT:
                    dy_copy((t + 1) % NT, (g + 1) % 2).start()
                dy_copy(t, slot).wait()
                mm = lax.dot_general(
                    xs, dy_buf[slot],
                    dimension_numbers=(((0,), (0,)), ((), ())),
                    preferred_element_type=jnp.float32,
                )
                fsl = pl.ds(t * FTILE, FTILE)
                if s == 0:
                    send_buf[:, fsl] = mm.astype(jnp.bfloat16)
                elif s < N_DEV - 1:
                    send_buf[:, fsl] = (
                        mm + recv_buf[s - 1, :, fsl].astype(jnp.float32)
                    ).astype(jnp.bfloat16)
                else:
                    out_ref[:, fsl] = (
                        mm + recv_buf[N_DEV - 2, :, fsl].astype(jnp.float32)
                    ).astype(jnp.bfloat16)

            if s < N_DEV - 1:
                rdma = pltpu.make_async_remote_copy(
                    src_ref=send_buf,
                    dst_ref=recv_buf.at[s],
                    send_sem=send_sems.at[s],
                    recv_sem=recv_sems.at[s],
                    device_id=(right,),
                    device_id_type=pl.DeviceIdType.MESH,
                )
                rdma.start()
                rdma.wait()

    return pl.pallas_call(
        body,
        out_shape=jax.ShapeDtypeStruct((CHUNK, F), jnp.bfloat16),
        in_specs=[
            pl.BlockSpec(memory_space=pltpu.VMEM),
            pl.BlockSpec(memory_space=pl.ANY),
        ],
        out_specs=pl.BlockSpec(memory_space=pltpu.VMEM),
        scratch_shapes=[
            pltpu.VMEM((2, M, FTILE), jnp.bfloat16),
            pltpu.VMEM((CHUNK, F), jnp.bfloat16),
            pltpu.VMEM((N_DEV - 1, CHUNK, F), jnp.bfloat16),
            pltpu.SemaphoreType.DMA((2,)),
            pltpu.SemaphoreType.DMA((N_DEV - 1,)),
            pltpu.SemaphoreType.DMA((N_DEV - 1,)),
        ],
        compiler_params=pltpu.CompilerParams(
            collective_id=0,
            vmem_limit_bytes=60 * 1024 * 1024,
        ),
    )(x, dy)


# device time: 283405 ns/iter; 1.4793x vs baseline; 1.4793x over previous
import jax
import jax.numpy as jnp
from jax import lax
from jax.experimental import pallas as pl
from jax.experimental.pallas import tpu as pltpu

N_DEV = 4
M = 2048
D = 2048
F = 8192
CHUNK = D // N_DEV
FTILE = 1024
NT = F // FTILE
HALF = NT // 2


def kernel(x, dy):
    x = x.astype(jnp.bfloat16)
    dy = dy.astype(jnp.bfloat16)

    def body(x_ref, dy_hbm, out_ref, dy_buf, send_buf, recv_buf,
             dy_sems, send_sems, recv_sems):
        my = lax.axis_index("i")
        left = lax.rem(my + N_DEV - 1, N_DEV)
        right = lax.rem(my + 1, N_DEV)

        barrier = pltpu.get_barrier_semaphore()
        for nbr in (left, right):
            pl.semaphore_signal(
                barrier, inc=1,
                device_id=(nbr,), device_id_type=pl.DeviceIdType.MESH,
            )
        pl.semaphore_wait(barrier, 2)

        def dy_copy(t, slot):
            return pltpu.make_async_copy(
                dy_hbm.at[:, pl.ds(t * FTILE, FTILE)],
                dy_buf.at[slot],
                dy_sems.at[slot],
            )

        dy_copy(0, 0).start()

        for s in range(N_DEV):
            if s < N_DEV - 1:
                c_r = lax.rem(my + 2 * N_DEV - 1 - s, N_DEV)
                c_l = lax.rem(my + 1 + s, N_DEV)
            else:
                c_r = c_l = my
            xs_r = x_ref[:, pl.ds(c_r * CHUNK, CHUNK)]
            xs_l = x_ref[:, pl.ds(c_l * CHUNK, CHUNK)]

            for t in range(NT):
                g = s * NT + t
                slot = g % 2
                if g + 1 < N_DEV * NT:
                    dy_copy((t + 1) % NT, (g + 1) % 2).start()
                dy_copy(t, slot).wait()
                mm = lax.dot_general(
                    xs_r if t < HALF else xs_l, dy_buf[slot],
                    dimension_numbers=(((0,), (0,)), ((), ())),
                    preferred_element_type=jnp.float32,
                )
                if s == 0:
                    send_buf[t] = mm.astype(jnp.bfloat16)
                elif s < N_DEV - 1:
                    send_buf[t] = (
                        mm + recv_buf[s - 1, t].astype(jnp.float32)
                    ).astype(jnp.bfloat16)
                else:
                    out_ref[:, pl.ds(t * FTILE, FTILE)] = (
                        mm + recv_buf[N_DEV - 2, t].astype(jnp.float32)
                    ).astype(jnp.bfloat16)

            if s < N_DEV - 1:
                rdmas = []
                for d, (lo, nbr) in enumerate(((0, right), (HALF, left))):
                    rdmas.append(pltpu.make_async_remote_copy(
                        src_ref=send_buf.at[pl.ds(lo, HALF)],
                        dst_ref=recv_buf.at[s, pl.ds(lo, HALF)],
                        send_sem=send_sems.at[s, d],
                        recv_sem=recv_sems.at[s, d],
                        device_id=(nbr,),
                        device_id_type=pl.DeviceIdType.MESH,
                    ))
                for r in rdmas:
                    r.start()
                for r in rdmas:
                    r.wait()

    return pl.pallas_call(
        body,
        out_shape=jax.ShapeDtypeStruct((CHUNK, F), jnp.bfloat16),
        in_specs=[
            pl.BlockSpec(memory_space=pltpu.VMEM),
            pl.BlockSpec(memory_space=pl.ANY),
        ],
        out_specs=pl.BlockSpec(memory_space=pltpu.VMEM),
        scratch_shapes=[
            pltpu.VMEM((2, M, FTILE), jnp.bfloat16),
            pltpu.VMEM((NT, CHUNK, FTILE), jnp.bfloat16),
            pltpu.VMEM((N_DEV - 1, NT, CHUNK, FTILE), jnp.bfloat16),
            pltpu.SemaphoreType.DMA((2,)),
            pltpu.SemaphoreType.DMA((N_DEV - 1, 2)),
            pltpu.SemaphoreType.DMA((N_DEV - 1, 2)),
        ],
        compiler_params=pltpu.CompilerParams(
            collective_id=0,
            vmem_limit_bytes=60 * 1024 * 1024,
        ),
    )(x, dy)


# device time: 206882 ns/iter; 2.0264x vs baseline; 1.3699x over previous
import jax
import jax.numpy as jnp
from jax import lax
from jax.experimental import pallas as pl
from jax.experimental.pallas import tpu as pltpu

N_DEV = 4
M = 2048
D = 2048
F = 8192
CHUNK = D // N_DEV
FTILE = 1024
NT = F // FTILE


def kernel(x, dy):
    x = x.astype(jnp.bfloat16)
    dy = dy.astype(jnp.bfloat16)

    def body(x_ref, dy_hbm, out_ref, dy_buf, send_buf, recv_buf,
             dy_sems, send_sems, recv_sems):
        my = lax.axis_index("i")
        left = lax.rem(my + N_DEV - 1, N_DEV)
        right = lax.rem(my + 1, N_DEV)

        barrier = pltpu.get_barrier_semaphore()
        for nbr in (left, right):
            pl.semaphore_signal(
                barrier, inc=1,
                device_id=(nbr,), device_id_type=pl.DeviceIdType.MESH,
            )
        pl.semaphore_wait(barrier, 2)

        def dy_copy(t, slot):
            return pltpu.make_async_copy(
                dy_hbm.at[:, pl.ds(t * FTILE, FTILE)],
                dy_buf.at[slot],
                dy_sems.at[slot],
            )

        def tile_rdma(s, t):
            return pltpu.make_async_remote_copy(
                src_ref=send_buf.at[t],
                dst_ref=recv_buf.at[s, t],
                send_sem=send_sems.at[s, t],
                recv_sem=recv_sems.at[s, t],
                device_id=(right if t % 2 == 0 else left,),
                device_id_type=pl.DeviceIdType.MESH,
            )

        dy_copy(0, 0).start()

        for s in range(N_DEV):
            if s < N_DEV - 1:
                c_r = lax.rem(my + 2 * N_DEV - 1 - s, N_DEV)
                c_l = lax.rem(my + 1 + s, N_DEV)
            else:
                c_r = c_l = my
            xs_r = x_ref[:, pl.ds(c_r * CHUNK, CHUNK)]
            xs_l = x_ref[:, pl.ds(c_l * CHUNK, CHUNK)]

            for t in range(NT):
                g = s * NT + t
                slot = g % 2
                if g + 1 < N_DEV * NT:
                    dy_copy((t + 1) % NT, (g + 1) % 2).start()
                dy_copy(t, slot).wait()
                if s >= 1:
                    tile_rdma(s - 1, t).wait_recv()
                    tile_rdma(s - 1, t).wait_send()
                mm = lax.dot_general(
                    xs_r if t % 2 == 0 else xs_l, dy_buf[slot],
                    dimension_numbers=(((0,), (0,)), ((), ())),
                    preferred_element_type=jnp.float32,
                )
                if s == 0:
                    send_buf[t] = mm.astype(jnp.bfloat16)
                elif s < N_DEV - 1:
                    send_buf[t] = (
                        mm + recv_buf[s - 1, t].astype(jnp.float32)
                    ).astype(jnp.bfloat16)
                else:
                    out_ref[:, pl.ds(t * FTILE, FTILE)] = (
                        mm + recv_buf[N_DEV - 2, t].astype(jnp.float32)
                    ).astype(jnp.bfloat16)
                if s < N_DEV - 1:
                    tile_rdma(s, t).start()

    return pl.pallas_call(
        body,
        out_shape=jax.ShapeDtypeStruct((CHUNK, F), jnp.bfloat16),
        in_specs=[
            pl.BlockSpec(memory_space=pltpu.VMEM),
            pl.BlockSpec(memory_space=pl.ANY),
        ],
        out_specs=pl.BlockSpec(memory_space=pltpu.VMEM),
        scratch_shapes=[
            pltpu.VMEM((2, M, FTILE), jnp.bfloat16),
            pltpu.VMEM((NT, CHUNK, FTILE), jnp.bfloat16),
            pltpu.VMEM((N_DEV - 1, NT, CHUNK, FTILE), jnp.bfloat16),
            pltpu.SemaphoreType.DMA((2,)),
            pltpu.SemaphoreType.DMA((N_DEV - 1, NT)),
            pltpu.SemaphoreType.DMA((N_DEV - 1, NT)),
        ],
        compiler_params=pltpu.CompilerParams(
            collective_id=0,
            vmem_limit_bytes=60 * 1024 * 1024,
        ),
    )(x, dy)


# device time: 174355 ns/iter; 2.4045x vs baseline; 1.1866x over previous
import jax
import jax.numpy as jnp
from jax import lax
from jax.experimental import pallas as pl
from jax.experimental.pallas import tpu as pltpu

N_DEV = 4
M = 2048
D = 2048
F = 8192
CHUNK = D // N_DEV
FTILE = 1024
NT = F // FTILE
FSUB = 512
NSUB = F // FSUB
SUB_PER_TILE = FTILE // FSUB


def kernel(x, dy):
    x = x.astype(jnp.bfloat16)

    def body(x_ref, dy_hbm, out_ref, dy_buf, send_buf, recv_buf,
             dy_sems, send_sems, recv_sems):
        my = lax.axis_index("i")
        left = lax.rem(my + N_DEV - 1, N_DEV)
        right = lax.rem(my + 1, N_DEV)

        barrier = pltpu.get_barrier_semaphore()
        for nbr in (left, right):
            pl.semaphore_signal(
                barrier, inc=1,
                device_id=(nbr,), device_id_type=pl.DeviceIdType.MESH,
            )
        pl.semaphore_wait(barrier, 2)

        def dy_copy(u, slot):
            return pltpu.make_async_copy(
                dy_hbm.at[:, pl.ds(u * FSUB, FSUB)],
                dy_buf.at[slot],
                dy_sems.at[slot],
            )

        def tile_rdma(s, t):
            return pltpu.make_async_remote_copy(
                src_ref=send_buf.at[t],
                dst_ref=recv_buf.at[s, t],
                send_sem=send_sems.at[s, t],
                recv_sem=recv_sems.at[s, t],
                device_id=(right if t % 2 == 0 else left,),
                device_id_type=pl.DeviceIdType.MESH,
            )

        dy_copy(0, 0).start()

        for s in range(N_DEV):
            if s < N_DEV - 1:
                c_r = lax.rem(my + 2 * N_DEV - 1 - s, N_DEV)
                c_l = lax.rem(my + 1 + s, N_DEV)
            else:
                c_r = c_l = my
            xs_r = x_ref[:, pl.ds(c_r * CHUNK, CHUNK)]
            xs_l = x_ref[:, pl.ds(c_l * CHUNK, CHUNK)]

            for u in range(NSUB):
                t, h = u // SUB_PER_TILE, u % SUB_PER_TILE
                g = s * NSUB + u
                slot = g % 2
                if g + 1 < N_DEV * NSUB:
                    dy_copy((u + 1) % NSUB, (g + 1) % 2).start()
                dy_copy(u, slot).wait()
                if s >= 1 and h == 0:
                    tile_rdma(s - 1, t).wait_recv()
                    tile_rdma(s - 1, t).wait_send()
                mm = lax.dot_general(
                    xs_r if t % 2 == 0 else xs_l,
                    dy_buf[slot].astype(jnp.bfloat16),
                    dimension_numbers=(((0,), (0,)), ((), ())),
                    preferred_element_type=jnp.float32,
                )
                hsl = pl.ds(h * FSUB, FSUB)
                if s == 0:
                    send_buf[t, :, hsl] = mm.astype(jnp.bfloat16)
                elif s < N_DEV - 1:
                    send_buf[t, :, hsl] = (
                        mm + recv_buf[s - 1, t, :, hsl].astype(jnp.float32)
                    ).astype(jnp.bfloat16)
                else:
                    out_ref[:, pl.ds(u * FSUB, FSUB)] = (
                        mm + recv_buf[N_DEV - 2, t, :, hsl].astype(jnp.float32)
                    ).astype(jnp.bfloat16)
                if s < N_DEV - 1 and h == SUB_PER_TILE - 1:
                    tile_rdma(s, t).start()

    return pl.pallas_call(
        body,
        out_shape=jax.ShapeDtypeStruct((CHUNK, F), jnp.bfloat16),
        in_specs=[
            pl.BlockSpec(memory_space=pltpu.VMEM),
            pl.BlockSpec(memory_space=pl.ANY),
        ],
        out_specs=pl.BlockSpec(memory_space=pltpu.VMEM),
        scratch_shapes=[
            pltpu.VMEM((2, M, FSUB), jnp.float32),
            pltpu.VMEM((NT, CHUNK, FTILE), jnp.bfloat16),
            pltpu.VMEM((N_DEV - 1, NT, CHUNK, FTILE), jnp.bfloat16),
            pltpu.SemaphoreType.DMA((2,)),
            pltpu.SemaphoreType.DMA((N_DEV - 1, NT)),
            pltpu.SemaphoreType.DMA((N_DEV - 1, NT)),
        ],
        compiler_params=pltpu.CompilerParams(
            collective_id=0,
            vmem_limit_bytes=60 * 1024 * 1024,
        ),
    )(x, dy)
